# baseline (speedup 1.0000x reference)
import functools

import jax
import jax.numpy as jnp
from jax import lax
from jax.experimental import pallas as pl
from jax.experimental.pallas import tpu as pltpu
from jax.experimental.pallas import tpu_sc as plsc

NUM_CODES = 8192
DIM = 32
N_ROWS = 16384
ROW_BLOCK = 128
NUM_WORKERS = 32
ROWS_PER_WORKER = N_ROWS // NUM_WORKERS
COMMIT = 0.25


CHUNK = 4096


def _tc_argmin_body(h_ref, h2_ref, w2_ref, iota_ref, w_ref, idx_ref):
    hb = (h_ref[...] * 2.0).astype(jnp.bfloat16)
    m2 = lax.dot_general(
        hb, w_ref[...], (((1,), (1,)), ((), ())),
        preferred_element_type=jnp.float32)
    d = (h2_ref[...] + w2_ref[...]) - m2
    acc_v = None
    acc_i = None
    for c in range(NUM_CODES // CHUNK):
        dc = d[:, c * CHUNK:(c + 1) * CHUNK]
        mn = jnp.min(dc, axis=1, keepdims=True)
        iota = iota_ref[:, c * CHUNK:(c + 1) * CHUNK]
        cand = jnp.where(dc == mn, iota, float(NUM_CODES))
        mi = jnp.min(cand, axis=1, keepdims=True)
        if acc_v is None:
            acc_v, acc_i = mn, mi
        else:
            take = (mn < acc_v) | ((mn == acc_v) & (mi < acc_i))
            acc_v = jnp.where(take, mn, acc_v)
            acc_i = jnp.where(take, mi, acc_i)
        acc_v = acc_v.astype(jnp.bfloat16).astype(jnp.float32)
    idx_ref[...] = acc_i.astype(jnp.int32)


def _tc_argmin_call(hidden, h2, w2, weight):
    grid = (N_ROWS // ROW_BLOCK,)
    return pl.pallas_call(
        _tc_argmin_body,
        grid=grid,
        in_specs=[
            pl.BlockSpec((ROW_BLOCK, DIM), lambda i: (i, 0)),
            pl.BlockSpec((ROW_BLOCK, 1), lambda i: (i, 0)),
            pl.BlockSpec((1, NUM_CODES), lambda i: (0, 0)),
            pl.BlockSpec((1, NUM_CODES), lambda i: (0, 0)),
            pl.BlockSpec((NUM_CODES, DIM), lambda i: (0, 0)),
        ],
        out_specs=pl.BlockSpec((ROW_BLOCK, 1), lambda i: (i, 0)),
        out_shape=jax.ShapeDtypeStruct((N_ROWS, 1), jnp.int32),
    )(hidden, h2, w2,
      jnp.arange(NUM_CODES, dtype=jnp.float32)[None, :],
      weight.astype(jnp.bfloat16))


def _sc_lookup_body(idx_hbm, w_hbm, x_hbm, qst_hbm, loss_hbm,
                    idx_v, rows_v, x_v, qst_v, loss_v, sem):
    wid = lax.axis_index("s") * 2 + lax.axis_index("c")
    base = wid * ROWS_PER_WORKER
    pltpu.sync_copy(idx_hbm.at[pl.ds(base, ROWS_PER_WORKER)], idx_v)
    gather = pltpu.async_copy(w_hbm.at[idx_v], rows_v, sem)
    pltpu.sync_copy(x_hbm.at[pl.ds(base, ROWS_PER_WORKER)], x_v)
    gather.wait()

    def body(i, carry):
        for h in range(DIM // 16):
            sl = (i, pl.ds(h * 16, 16))
            q = rows_v[sl]
            x = x_v[sl]
            dlt = q - x
            qst_v[sl] = x + dlt
            d2 = dlt * dlt
            loss_v[sl] = d2 + COMMIT * d2
        return carry

    lax.fori_loop(0, ROWS_PER_WORKER, body, 0)
    pltpu.sync_copy(qst_v, qst_hbm.at[pl.ds(base, ROWS_PER_WORKER)])
    pltpu.sync_copy(loss_v, loss_hbm.at[pl.ds(base, ROWS_PER_WORKER)])


_sc_lookup = functools.partial(
    pl.kernel,
    mesh=plsc.VectorSubcoreMesh(core_axis_name="c", subcore_axis_name="s"),
    compiler_params=pltpu.CompilerParams(use_tc_tiling_on_sc=False),
    out_type=(
        jax.ShapeDtypeStruct((N_ROWS, DIM), jnp.float32),
        jax.ShapeDtypeStruct((N_ROWS, DIM), jnp.float32),
    ),
    scratch_types=[
        pltpu.VMEM((ROWS_PER_WORKER,), jnp.int32),
        pltpu.VMEM((ROWS_PER_WORKER, DIM), jnp.float32),
        pltpu.VMEM((ROWS_PER_WORKER, DIM), jnp.float32),
        pltpu.VMEM((ROWS_PER_WORKER, DIM), jnp.float32),
        pltpu.VMEM((ROWS_PER_WORKER, DIM), jnp.float32),
        pltpu.SemaphoreType.DMA,
    ],
)(_sc_lookup_body)


def kernel(input, weight):
    shape = input.shape
    hidden = input.reshape(-1, DIM)
    h2 = jnp.sum(hidden ** 2, axis=1, keepdims=True)
    w2 = jnp.sum(weight ** 2, axis=1)[None, :]
    idx = _tc_argmin_call(hidden, h2, w2, weight)
    w_lookup = weight.astype(jnp.bfloat16).astype(jnp.float32)
    qst, loss = _sc_lookup(jnp.squeeze(idx, -1), w_lookup, hidden)
    return qst.reshape(shape), loss.reshape(shape)


# speedup vs baseline: 1.2088x; 1.2088x over previous
import functools

import jax
import jax.numpy as jnp
from jax import lax
from jax.experimental import pallas as pl
from jax.experimental.pallas import tpu as pltpu
from jax.experimental.pallas import tpu_sc as plsc

NUM_CODES = 8192
DIM = 32
N_ROWS = 16384
ROW_BLOCK = 256
NUM_WORKERS = 32
ROWS_PER_WORKER = N_ROWS // NUM_WORKERS
COMMIT = 0.25


CHUNK = 4096


def _tc_argmin_body(h_ref, h2_ref, w2_ref, iota_ref, w_ref, idx_ref):
    hb = h_ref[...].astype(jnp.bfloat16)
    m = lax.dot_general(
        hb, w_ref[...], (((1,), (1,)), ((), ())),
        preferred_element_type=jnp.float32)
    d = (h2_ref[...] + w2_ref[...]) - 2.0 * m
    acc_v = None
    acc_i = None
    for c in range(NUM_CODES // CHUNK):
        dc = d[:, c * CHUNK:(c + 1) * CHUNK]
        mn = jnp.min(dc, axis=1, keepdims=True)
        iota = iota_ref[:, c * CHUNK:(c + 1) * CHUNK]
        cand = jnp.where(dc == mn, iota, float(NUM_CODES))
        mi = jnp.min(cand, axis=1, keepdims=True)
        if acc_v is None:
            acc_v, acc_i = mn, mi
        else:
            take = (mn < acc_v) | ((mn == acc_v) & (mi < acc_i))
            acc_v = jnp.where(take, mn, acc_v)
            acc_i = jnp.where(take, mi, acc_i)
        acc_v = acc_v.astype(jnp.bfloat16).astype(jnp.float32)
    idx_ref[...] = acc_i.astype(jnp.int32)


def _tc_argmin_call(hidden, h2, w2, weight):
    grid = (N_ROWS // ROW_BLOCK,)
    return pl.pallas_call(
        _tc_argmin_body,
        grid=grid,
        in_specs=[
            pl.BlockSpec((ROW_BLOCK, DIM), lambda i: (i, 0)),
            pl.BlockSpec((ROW_BLOCK, 1), lambda i: (i, 0)),
            pl.BlockSpec((1, NUM_CODES), lambda i: (0, 0)),
            pl.BlockSpec((1, NUM_CODES), lambda i: (0, 0)),
            pl.BlockSpec((NUM_CODES, DIM), lambda i: (0, 0)),
        ],
        out_specs=pl.BlockSpec((ROW_BLOCK, 1), lambda i: (i, 0)),
        out_shape=jax.ShapeDtypeStruct((N_ROWS, 1), jnp.int32),
    )(hidden, h2, w2,
      jnp.arange(NUM_CODES, dtype=jnp.float32)[None, :],
      weight.astype(jnp.bfloat16))


def _sc_lookup_body(idx_hbm, w_hbm, x_hbm, qst_hbm, loss_hbm,
                    idx_v, rows_v, x_v, qst_v, loss_v, sem):
    wid = lax.axis_index("s") * 2 + lax.axis_index("c")
    base = wid * ROWS_PER_WORKER
    pltpu.sync_copy(idx_hbm.at[pl.ds(base, ROWS_PER_WORKER)], idx_v)
    gather = pltpu.async_copy(w_hbm.at[idx_v], rows_v, sem)
    pltpu.sync_copy(x_hbm.at[pl.ds(base, ROWS_PER_WORKER)], x_v)
    gather.wait()

    def body(i, carry):
        for h in range(DIM // 16):
            sl = (i, pl.ds(h * 16, 16))
            q = rows_v[sl]
            x = x_v[sl]
            dlt = q - x
            qst_v[sl] = x + dlt
            d2 = dlt * dlt
            loss_v[sl] = d2 + COMMIT * d2
        return carry

    lax.fori_loop(0, ROWS_PER_WORKER, body, 0)
    pltpu.sync_copy(qst_v, qst_hbm.at[pl.ds(base, ROWS_PER_WORKER)])
    pltpu.sync_copy(loss_v, loss_hbm.at[pl.ds(base, ROWS_PER_WORKER)])


_sc_lookup = functools.partial(
    pl.kernel,
    mesh=plsc.VectorSubcoreMesh(core_axis_name="c", subcore_axis_name="s"),
    compiler_params=pltpu.CompilerParams(use_tc_tiling_on_sc=False),
    out_type=(
        jax.ShapeDtypeStruct((N_ROWS, DIM), jnp.float32),
        jax.ShapeDtypeStruct((N_ROWS, DIM), jnp.float32),
    ),
    scratch_types=[
        pltpu.VMEM((ROWS_PER_WORKER,), jnp.int32),
        pltpu.VMEM((ROWS_PER_WORKER, DIM), jnp.float32),
        pltpu.VMEM((ROWS_PER_WORKER, DIM), jnp.float32),
        pltpu.VMEM((ROWS_PER_WORKER, DIM), jnp.float32),
        pltpu.VMEM((ROWS_PER_WORKER, DIM), jnp.float32),
        pltpu.SemaphoreType.DMA,
    ],
)(_sc_lookup_body)


def kernel(input, weight):
    shape = input.shape
    hidden = input.reshape(-1, DIM)
    h2 = jnp.sum(hidden ** 2, axis=1, keepdims=True)
    w2 = jnp.sum(weight ** 2, axis=1)[None, :]
    idx = _tc_argmin_call(hidden, h2, w2, weight)
    w_lookup = weight.astype(jnp.bfloat16).astype(jnp.float32)
    qst, loss = _sc_lookup(jnp.squeeze(idx, -1), w_lookup, hidden)
    return qst.reshape(shape), loss.reshape(shape)


# speedup vs baseline: 1.2456x; 1.0305x over previous
import functools

import jax
import jax.numpy as jnp
from jax import lax
from jax.experimental import pallas as pl
from jax.experimental.pallas import tpu as pltpu
from jax.experimental.pallas import tpu_sc as plsc

NUM_CODES = 8192
DIM = 32
N_ROWS = 16384
ROW_BLOCK = 512
NUM_WORKERS = 32
ROWS_PER_WORKER = N_ROWS // NUM_WORKERS
COMMIT = 0.25


CHUNK = 4096


def _tc_argmin_body(h_ref, h2_ref, w2_ref, iota_ref, w_ref, idx_ref):
    hb = h_ref[...].astype(jnp.bfloat16)
    m = lax.dot_general(
        hb, w_ref[...], (((1,), (1,)), ((), ())),
        preferred_element_type=jnp.float32)
    d = (h2_ref[...] + w2_ref[...]) - 2.0 * m
    acc_v = None
    acc_i = None
    for c in range(NUM_CODES // CHUNK):
        dc = d[:, c * CHUNK:(c + 1) * CHUNK]
        mn = jnp.min(dc, axis=1, keepdims=True)
        iota = iota_ref[:, c * CHUNK:(c + 1) * CHUNK]
        cand = jnp.where(dc == mn, iota, float(NUM_CODES))
        mi = jnp.min(cand, axis=1, keepdims=True)
        if acc_v is None:
            acc_v, acc_i = mn, mi
        else:
            take = (mn < acc_v) | ((mn == acc_v) & (mi < acc_i))
            acc_v = jnp.where(take, mn, acc_v)
            acc_i = jnp.where(take, mi, acc_i)
        acc_v = acc_v.astype(jnp.bfloat16).astype(jnp.float32)
    idx_ref[...] = acc_i.astype(jnp.int32)


def _tc_argmin_call(hidden, h2, w2, weight):
    grid = (N_ROWS // ROW_BLOCK,)
    return pl.pallas_call(
        _tc_argmin_body,
        grid=grid,
        in_specs=[
            pl.BlockSpec((ROW_BLOCK, DIM), lambda i: (i, 0)),
            pl.BlockSpec((ROW_BLOCK, 1), lambda i: (i, 0)),
            pl.BlockSpec((1, NUM_CODES), lambda i: (0, 0)),
            pl.BlockSpec((1, NUM_CODES), lambda i: (0, 0)),
            pl.BlockSpec((NUM_CODES, DIM), lambda i: (0, 0)),
        ],
        out_specs=pl.BlockSpec((ROW_BLOCK, 1), lambda i: (i, 0)),
        out_shape=jax.ShapeDtypeStruct((N_ROWS, 1), jnp.int32),
    )(hidden, h2, w2,
      jnp.arange(NUM_CODES, dtype=jnp.float32)[None, :],
      weight.astype(jnp.bfloat16))


def _sc_lookup_body(idx_hbm, w_hbm, x_hbm, qst_hbm, loss_hbm,
                    idx_v, rows_v, x_v, qst_v, loss_v, sem):
    wid = lax.axis_index("s") * 2 + lax.axis_index("c")
    base = wid * ROWS_PER_WORKER
    pltpu.sync_copy(idx_hbm.at[pl.ds(base, ROWS_PER_WORKER)], idx_v)
    gather = pltpu.async_copy(w_hbm.at[idx_v], rows_v, sem)
    pltpu.sync_copy(x_hbm.at[pl.ds(base, ROWS_PER_WORKER)], x_v)
    gather.wait()

    def body(i, carry):
        for h in range(DIM // 16):
            sl = (i, pl.ds(h * 16, 16))
            q = rows_v[sl]
            x = x_v[sl]
            dlt = q - x
            qst_v[sl] = x + dlt
            d2 = dlt * dlt
            loss_v[sl] = d2 + COMMIT * d2
        return carry

    lax.fori_loop(0, ROWS_PER_WORKER, body, 0)
    pltpu.sync_copy(qst_v, qst_hbm.at[pl.ds(base, ROWS_PER_WORKER)])
    pltpu.sync_copy(loss_v, loss_hbm.at[pl.ds(base, ROWS_PER_WORKER)])


_sc_lookup = functools.partial(
    pl.kernel,
    mesh=plsc.VectorSubcoreMesh(core_axis_name="c", subcore_axis_name="s"),
    compiler_params=pltpu.CompilerParams(use_tc_tiling_on_sc=False),
    out_type=(
        jax.ShapeDtypeStruct((N_ROWS, DIM), jnp.float32),
        jax.ShapeDtypeStruct((N_ROWS, DIM), jnp.float32),
    ),
    scratch_types=[
        pltpu.VMEM((ROWS_PER_WORKER,), jnp.int32),
        pltpu.VMEM((ROWS_PER_WORKER, DIM), jnp.float32),
        pltpu.VMEM((ROWS_PER_WORKER, DIM), jnp.float32),
        pltpu.VMEM((ROWS_PER_WORKER, DIM), jnp.float32),
        pltpu.VMEM((ROWS_PER_WORKER, DIM), jnp.float32),
        pltpu.SemaphoreType.DMA,
    ],
)(_sc_lookup_body)


def kernel(input, weight):
    shape = input.shape
    hidden = input.reshape(-1, DIM)
    h2 = jnp.sum(hidden ** 2, axis=1, keepdims=True)
    w2 = jnp.sum(weight ** 2, axis=1)[None, :]
    idx = _tc_argmin_call(hidden, h2, w2, weight)
    w_lookup = weight.astype(jnp.bfloat16).astype(jnp.float32)
    qst, loss = _sc_lookup(jnp.squeeze(idx, -1), w_lookup, hidden)
    return qst.reshape(shape), loss.reshape(shape)
